# baseline (speedup 1.0000x reference)
import functools

import jax
import jax.numpy as jnp
from jax import lax
from jax.experimental import pallas as pl
from jax.experimental.pallas import tpu as pltpu
from jax.experimental.pallas import tpu_sc as plsc

VOCAB = 100000
HIDDEN = 128
BATCH = 4096
CTX = 20

NUM_CORES = 2
NUM_SUBCORES = 16
NUM_WORKERS = NUM_CORES * NUM_SUBCORES
BPW = BATCH // NUM_WORKERS
LANES = 16
HCHUNKS = HIDDEN // LANES


def _sc_gather_sum(idx_t, emb_table):
    mesh = plsc.VectorSubcoreMesh(core_axis_name="c", subcore_axis_name="s")

    @functools.partial(
        pl.kernel,
        out_type=jax.ShapeDtypeStruct((BATCH, HIDDEN), jnp.float32),
        mesh=mesh,
        scratch_types=[
            pltpu.VMEM((CTX, BPW), jnp.int32),
            pltpu.VMEM((2, BPW, HIDDEN), jnp.float32),
            pltpu.VMEM((BPW, HIDDEN), jnp.float32),
            pltpu.SemaphoreType.DMA,
            pltpu.SemaphoreType.DMA,
            pltpu.SemaphoreType.DMA,
        ],
    )
    def k(idx_hbm, table_hbm, out_hbm, idx_v, rows_v, acc_v, sem0, semA, semB):
        wid = lax.axis_index("s") * NUM_CORES + lax.axis_index("c")
        base = wid * BPW
        pltpu.sync_copy(idx_hbm.at[:, pl.ds(base, BPW)], idx_v)
        sems = (semA, semB)
        cp0 = pltpu.async_copy(table_hbm.at[idx_v.at[0]], acc_v, sem0)
        pending = pltpu.async_copy(
            table_hbm.at[idx_v.at[1]], rows_v.at[0], sems[0])
        cp0.wait()
        for c in range(1, CTX):
            buf = (c - 1) % 2
            if c + 1 < CTX:
                nxt = pltpu.async_copy(
                    table_hbm.at[idx_v.at[c + 1]], rows_v.at[c % 2],
                    sems[c % 2])
            pending.wait()
            if c + 1 < CTX:
                pending = nxt

            @plsc.parallel_loop(0, BPW, 1, unroll=2)
            def row_step(i):
                for h in range(HCHUNKS):
                    sl = pl.ds(h * LANES, LANES)
                    plsc.addupdate(acc_v.at[i, sl], rows_v[buf, i, sl])

        pltpu.sync_copy(acc_v, out_hbm.at[pl.ds(base, BPW)])

    return k(idx_t, emb_table)


BM = 1024
BN = 2048
GI = BATCH // BM
GJM = VOCAB // BN
NBUF = 4
LAST = GI * GJM - 1
GJ = GJM + 1
SBM = 256


def _ring_desc(o_hbm, o_buf, sems, s, i, j):
    return pltpu.make_async_copy(
        o_buf.at[s],
        o_hbm.at[pl.ds(i * BM, BM), pl.ds(j * BN, BN)],
        sems.at[s],
    )


def _stitch_inner(s_ref, w_ref, b_ref, o_ref):
    o_ref[...] = lax.dot_general(
        s_ref[...],
        w_ref[...],
        (((1,), (1,)), ((), ())),
        preferred_element_type=jnp.float32,
    ) + b_ref[0].astype(jnp.float32)


def _mm_body(s_ref, w_ref, b_ref, s_any, w_any, b_any, o_hbm, o_buf, sems):
    i = pl.program_id(0)
    j = pl.program_id(1)
    step = i * GJM + j
    slot = lax.rem(step, NBUF)

    acc = lax.dot_general(
        s_ref[...],
        w_ref[...],
        (((1,), (1,)), ((), ())),
        preferred_element_type=jnp.float32,
    ) + b_ref[0].astype(jnp.float32)

    @pl.when(step >= NBUF)
    def _():
        _ring_desc(o_hbm, o_buf, sems, slot, i, j).wait()

    o_buf[slot] = acc
    _ring_desc(o_hbm, o_buf, sems, slot, i, j).start()

    @pl.when(step == LAST)
    def _():
        for st in range(LAST - NBUF + 1, LAST + 1):
            _ring_desc(o_hbm, o_buf, sems, st % NBUF, i, j).wait()
        pltpu.emit_pipeline(
            _stitch_inner,
            grid=(BATCH // SBM,),
            in_specs=[
                pl.BlockSpec((SBM, HIDDEN), lambda b: (b, 0)),
                pl.BlockSpec((BN, HIDDEN), lambda b: (GJM, 0)),
                pl.BlockSpec((1, 1, BN), lambda b: (GJM, 0, 0)),
            ],
            out_specs=[pl.BlockSpec((SBM, BN), lambda b: (b, GJM))],
        )(s_any, w_any, b_any, o_hbm)


def _tc_matmul(summed, out_w, out_bp):
    return pl.pallas_call(
        _mm_body,
        grid=(GI, GJM),
        in_specs=[
            pl.BlockSpec((BM, HIDDEN), lambda i, j: (i, 0)),
            pl.BlockSpec((BN, HIDDEN), lambda i, j: (j, 0)),
            pl.BlockSpec((1, 1, BN), lambda i, j: (j, 0, 0)),
            pl.BlockSpec(memory_space=pl.ANY),
            pl.BlockSpec(memory_space=pl.ANY),
            pl.BlockSpec(memory_space=pl.ANY),
        ],
        out_specs=pl.BlockSpec(memory_space=pltpu.HBM),
        out_shape=jax.ShapeDtypeStruct((BATCH, VOCAB), jnp.float32),
        scratch_shapes=[
            pltpu.VMEM((NBUF, BM, BN), jnp.float32),
            pltpu.SemaphoreType.DMA((NBUF,)),
        ],
        compiler_params=pltpu.CompilerParams(
            dimension_semantics=("arbitrary", "arbitrary"),
        ),
    )(summed, out_w, out_bp, summed, out_w, out_bp)


def kernel(inputs, emb_table, out_w, out_b):
    idx_t = inputs.T.reshape(CTX, BATCH)
    summed = _sc_gather_sum(idx_t, emb_table)
    summed_bf = summed.astype(jnp.bfloat16)
    w_bf = out_w.astype(jnp.bfloat16)
    out_bp = jnp.pad(out_b, (0, GJ * BN - VOCAB)).reshape(GJ, 1, BN)
    return _tc_matmul(summed_bf, w_bf, out_bp)


# speedup vs baseline: 1.0177x; 1.0177x over previous
import functools

import jax
import jax.numpy as jnp
from jax import lax
from jax.experimental import pallas as pl
from jax.experimental.pallas import tpu as pltpu
from jax.experimental.pallas import tpu_sc as plsc

VOCAB = 100000
HIDDEN = 128
BATCH = 4096
CTX = 20

NUM_CORES = 2
NUM_SUBCORES = 16
NUM_WORKERS = NUM_CORES * NUM_SUBCORES
BPW = BATCH // NUM_WORKERS
LANES = 16
HCHUNKS = HIDDEN // LANES


def _sc_gather_sum(idx_t, emb_table):
    mesh = plsc.VectorSubcoreMesh(core_axis_name="c", subcore_axis_name="s")

    @functools.partial(
        pl.kernel,
        out_type=jax.ShapeDtypeStruct((BATCH, HIDDEN), jnp.float32),
        mesh=mesh,
        scratch_types=[
            pltpu.VMEM((CTX, BPW), jnp.int32),
            pltpu.VMEM((2, BPW, HIDDEN), jnp.float32),
            pltpu.VMEM((BPW, HIDDEN), jnp.float32),
            pltpu.SemaphoreType.DMA,
            pltpu.SemaphoreType.DMA,
            pltpu.SemaphoreType.DMA,
        ],
    )
    def k(idx_hbm, table_hbm, out_hbm, idx_v, rows_v, acc_v, sem0, semA, semB):
        wid = lax.axis_index("s") * NUM_CORES + lax.axis_index("c")
        base = wid * BPW
        pltpu.sync_copy(idx_hbm.at[:, pl.ds(base, BPW)], idx_v)
        sems = (semA, semB)
        cp0 = pltpu.async_copy(table_hbm.at[idx_v.at[0]], acc_v, sem0)
        pending = pltpu.async_copy(
            table_hbm.at[idx_v.at[1]], rows_v.at[0], sems[0])
        cp0.wait()
        for c in range(1, CTX):
            buf = (c - 1) % 2
            if c + 1 < CTX:
                nxt = pltpu.async_copy(
                    table_hbm.at[idx_v.at[c + 1]], rows_v.at[c % 2],
                    sems[c % 2])
            pending.wait()
            if c + 1 < CTX:
                pending = nxt

            @plsc.parallel_loop(0, BPW, 1, unroll=2)
            def row_step(i):
                for h in range(HCHUNKS):
                    sl = pl.ds(h * LANES, LANES)
                    plsc.addupdate(acc_v.at[i, sl], rows_v[buf, i, sl])

        pltpu.sync_copy(acc_v, out_hbm.at[pl.ds(base, BPW)])

    return k(idx_t, emb_table)


BM = 512
BN = 8192
GI = BATCH // BM
GJ = (VOCAB + BN - 1) // BN


def _mm_body(s_ref, w_ref, b_ref, o_ref):
    o_ref[...] = lax.dot_general(
        s_ref[...].astype(jnp.bfloat16),
        w_ref[...].astype(jnp.bfloat16),
        (((1,), (1,)), ((), ())),
        preferred_element_type=jnp.float32,
    ) + b_ref[0]


def _tc_matmul(summed, out_w, out_bp):
    return pl.pallas_call(
        _mm_body,
        grid=(GJ, GI),
        in_specs=[
            pl.BlockSpec((BM, HIDDEN), lambda j, i: (i, 0)),
            pl.BlockSpec((BN, HIDDEN), lambda j, i: (j, 0)),
            pl.BlockSpec((1, 1, BN), lambda j, i: (j, 0, 0)),
        ],
        out_specs=pl.BlockSpec((BM, BN), lambda j, i: (i, j)),
        out_shape=jax.ShapeDtypeStruct((BATCH, VOCAB), jnp.float32),
        compiler_params=pltpu.CompilerParams(
            dimension_semantics=("arbitrary", "arbitrary"),
        ),
    )(summed, out_w, out_bp)


def kernel(inputs, emb_table, out_w, out_b):
    idx_t = inputs.T.reshape(CTX, BATCH)
    summed = _sc_gather_sum(idx_t, emb_table)
    out_bp = jnp.pad(out_b, (0, GJ * BN - VOCAB)).reshape(GJ, 1, BN)
    return _tc_matmul(summed, out_w, out_bp)


# speedup vs baseline: 3.4225x; 3.3631x over previous
import functools

import jax
import jax.numpy as jnp
from jax import lax
from jax.experimental import pallas as pl
from jax.experimental.pallas import tpu as pltpu
from jax.experimental.pallas import tpu_sc as plsc

VOCAB = 100000
HIDDEN = 128
BATCH = 4096
CTX = 20

NUM_CORES = 2
NUM_SUBCORES = 16
NUM_WORKERS = NUM_CORES * NUM_SUBCORES
BPW = BATCH // NUM_WORKERS
LANES = 16
HCHUNKS = HIDDEN // LANES


def _sc_gather_sum(idx_t, emb_table):
    mesh = plsc.VectorSubcoreMesh(core_axis_name="c", subcore_axis_name="s")

    @functools.partial(
        pl.kernel,
        out_type=jax.ShapeDtypeStruct((BATCH, HIDDEN), jnp.float32),
        mesh=mesh,
        scratch_types=[
            pltpu.VMEM((CTX, BPW), jnp.int32),
            pltpu.VMEM((2, BPW, HIDDEN), jnp.float32),
            pltpu.VMEM((BPW, HIDDEN), jnp.float32),
            pltpu.SemaphoreType.DMA,
            pltpu.SemaphoreType.DMA,
            pltpu.SemaphoreType.DMA,
        ],
    )
    def k(idx_hbm, table_hbm, out_hbm, idx_v, rows_v, acc_v, sem0, semA, semB):
        wid = lax.axis_index("s") * NUM_CORES + lax.axis_index("c")
        base = wid * BPW
        pltpu.sync_copy(idx_hbm.at[:, pl.ds(base, BPW)], idx_v)
        sems = (semA, semB)
        cp0 = pltpu.async_copy(table_hbm.at[idx_v.at[0]], acc_v, sem0)
        pending = pltpu.async_copy(
            table_hbm.at[idx_v.at[1]], rows_v.at[0], sems[0])
        cp0.wait()
        for c in range(1, CTX):
            buf = (c - 1) % 2
            if c + 1 < CTX:
                nxt = pltpu.async_copy(
                    table_hbm.at[idx_v.at[c + 1]], rows_v.at[c % 2],
                    sems[c % 2])
            pending.wait()
            if c + 1 < CTX:
                pending = nxt

            @plsc.parallel_loop(0, BPW, 1, unroll=2)
            def row_step(i):
                for h in range(HCHUNKS):
                    sl = pl.ds(h * LANES, LANES)
                    plsc.addupdate(acc_v.at[i, sl], rows_v[buf, i, sl])

        pltpu.sync_copy(acc_v, out_hbm.at[pl.ds(base, BPW)])

    return k(idx_t, emb_table)


BMV = 800
GV = VOCAB // BMV
NBUF = 3
LAST = GV - 1


def _ring_desc(o_hbm, o_buf, sems, s, v):
    return pltpu.make_async_copy(
        o_buf.at[s],
        o_hbm.at[pl.ds(v * BMV, BMV)],
        sems.at[s],
    )


def _mm_body(w_ref, s_ref, b_ref, o_hbm, o_buf, sems):
    v = pl.program_id(0)
    slot = lax.rem(v, NBUF)

    acc = lax.dot_general(
        w_ref[...].astype(jnp.bfloat16),
        s_ref[...],
        (((1,), (1,)), ((), ())),
        preferred_element_type=jnp.float32,
    ) + b_ref[...]

    @pl.when(v >= NBUF)
    def _():
        _ring_desc(o_hbm, o_buf, sems, slot, v).wait()

    o_buf[slot] = acc
    _ring_desc(o_hbm, o_buf, sems, slot, v).start()

    @pl.when(v == LAST)
    def _():
        for st in range(LAST - NBUF + 1, LAST + 1):
            _ring_desc(o_hbm, o_buf, sems, st % NBUF, v).wait()


def _tc_matmul_t(out_w, summed_bf, out_b2):
    return pl.pallas_call(
        _mm_body,
        grid=(GV,),
        in_specs=[
            pl.BlockSpec((BMV, HIDDEN), lambda v: (v, 0)),
            pl.BlockSpec((BATCH, HIDDEN), lambda v: (0, 0)),
            pl.BlockSpec((BMV, 1), lambda v: (v, 0)),
        ],
        out_specs=pl.BlockSpec(memory_space=pl.ANY),
        out_shape=jax.ShapeDtypeStruct((VOCAB, BATCH), jnp.float32),
        scratch_shapes=[
            pltpu.VMEM((NBUF, BMV, BATCH), jnp.float32),
            pltpu.SemaphoreType.DMA((NBUF,)),
        ],
        compiler_params=pltpu.CompilerParams(
            dimension_semantics=("arbitrary",),
        ),
    )(out_w, summed_bf, out_b2)


def kernel(inputs, emb_table, out_w, out_b):
    idx_t = inputs.T.reshape(CTX, BATCH)
    summed = _sc_gather_sum(idx_t, emb_table)
    logits_t = _tc_matmul_t(
        out_w,
        summed.astype(jnp.bfloat16),
        out_b.reshape(VOCAB, 1),
    )
    return logits_t.T
